# baseline (speedup 1.0000x reference)
import jax
import jax.numpy as jnp
from jax.experimental import pallas as pl
from jax.experimental.pallas import tpu as pltpu

N = 10000
NFEAT = 128
H1 = 64
H2 = 32

CH = 200
NCH = N // CH
K = 5
CW = 1024
NT = (N + CW - 1) // CW

_PARAMS = pltpu.CompilerParams(
    dimension_semantics=(),
    vmem_limit_bytes=64 * 1024 * 1024,
)


def _chunk_copy(adj_ref, abuf_ref, sem_ref, c, slot):
    return pltpu.make_async_copy(
        adj_ref.at[pl.ds(c * CH, CH), :], abuf_ref.at[slot], sem_ref.at[slot]
    )


def _tile_copy(adj_ref, abuf_ref, sem_ref, c, slot, t):
    w = min(CW, N - t * CW)
    return pltpu.make_async_copy(
        adj_ref.at[pl.ds(c * CH, CH), pl.ds(t * CW, w)],
        abuf_ref.at[slot, :, pl.ds(t * CW, w)],
        sem_ref.at[slot],
    )


def _gcn_body(
    x_ref, adj_ref, w1_ref, b1_ref, w2_ref, b2_ref, o_ref, s1_ref, s2_ref,
    abuf_ref, sema_ref, semb_ref,
):
    for k in range(K):
        c0 = NCH - 1 - k
        _chunk_copy(adj_ref, abuf_ref, sema_ref, c0, c0 % K).start()

    s1_ref[...] = jnp.dot(x_ref[...], w1_ref[...], preferred_element_type=jnp.float32)
    s2_ref[...] = jnp.zeros((N, H2), jnp.float32)

    def p1_body(it, _):
        c = NCH - 1 - it
        slot = jax.lax.rem(c, K)
        _chunk_copy(adj_ref, abuf_ref, sema_ref, c, slot).wait()
        h = jnp.dot(abuf_ref[slot], s1_ref[...], preferred_element_type=jnp.float32)
        h = jnp.maximum(h + b1_ref[...], 0.0)
        s2_ref[pl.ds(c * CH, CH), :] = jnp.dot(
            h, w2_ref[...], preferred_element_type=jnp.float32
        )
        o_ref[pl.ds(c * CH, CH), :] = (
            jnp.dot(abuf_ref[slot], s2_ref[...], preferred_element_type=jnp.float32)
            + b2_ref[...]
        )

        @pl.when(c >= K)
        def _():
            _chunk_copy(adj_ref, abuf_ref, sema_ref, c - K, slot).start()

        return 0

    jax.lax.fori_loop(0, NCH, p1_body, 0)

    row_ids = jax.lax.broadcasted_iota(jnp.int32, (N, H2), 0)

    def fetch_tiles(c, slot):
        ntc = (c * CH + CW - 1) // CW
        for t in range(NT):
            @pl.when(t < ntc)
            def _():
                _tile_copy(adj_ref, abuf_ref, semb_ref, c, slot, t).start()

    def wait_tiles(c, slot):
        ntc = (c * CH + CW - 1) // CW
        for t in range(NT):
            @pl.when(t < ntc)
            def _():
                _tile_copy(adj_ref, abuf_ref, semb_ref, c, slot, t).wait()

    def p2_body(c, _):
        slot = jax.lax.rem(c, K)

        @pl.when(c >= K)
        def _():
            wait_tiles(c, slot)

        s2m = jnp.where(row_ids < c * CH, s2_ref[...], 0.0)
        o_ref[pl.ds(c * CH, CH), :] = o_ref[pl.ds(c * CH, CH), :] + jnp.dot(
            abuf_ref[slot], s2m, preferred_element_type=jnp.float32
        )

        @pl.when(c + K < NCH)
        def _():
            fetch_tiles(c + K, slot)

        return 0

    jax.lax.fori_loop(0, NCH, p2_body, 0)


@jax.jit
def _gcn(x, adj, W1, b1, W2, b2):
    b1r = b1.reshape(1, H1)
    b2r = b2.reshape(1, H2)

    out = pl.pallas_call(
        _gcn_body,
        in_specs=[
            pl.BlockSpec(memory_space=pltpu.MemorySpace.VMEM),
            pl.BlockSpec(memory_space=pl.ANY),
            pl.BlockSpec(memory_space=pltpu.MemorySpace.VMEM),
            pl.BlockSpec(memory_space=pltpu.MemorySpace.VMEM),
            pl.BlockSpec(memory_space=pltpu.MemorySpace.VMEM),
            pl.BlockSpec(memory_space=pltpu.MemorySpace.VMEM),
        ],
        out_specs=pl.BlockSpec(memory_space=pltpu.MemorySpace.VMEM),
        out_shape=jax.ShapeDtypeStruct((N, H2), jnp.float32),
        scratch_shapes=[
            pltpu.VMEM((N, H1), jnp.float32),
            pltpu.VMEM((N, H2), jnp.float32),
            pltpu.VMEM((K, CH, N), jnp.float32),
            pltpu.SemaphoreType.DMA((K,)),
            pltpu.SemaphoreType.DMA((K,)),
        ],
        compiler_params=_PARAMS,
    )(x, adj, W1, b1r, W2, b2r)

    return out


def kernel(x, adj, W1, b1, W2, b2):
    return _gcn(x, adj, W1, b1, W2, b2)


# speedup vs baseline: 1.3748x; 1.3748x over previous
import jax
import jax.numpy as jnp
from jax.experimental import pallas as pl
from jax.experimental.pallas import tpu as pltpu

N = 10000
NFEAT = 128
H1 = 64
H2 = 32

CH = 200
NCH = N // CH
K = 4
CW = 1024
NT = (N + CW - 1) // CW

_PARAMS = pltpu.CompilerParams(
    dimension_semantics=(),
    vmem_limit_bytes=64 * 1024 * 1024,
)


def _chunk_copy(adj_ref, abuf_ref, sem_ref, c, slot):
    return pltpu.make_async_copy(
        adj_ref.at[pl.ds(c * CH, CH), :], abuf_ref.at[slot], sem_ref.at[slot]
    )


def _tile_copy(adj_ref, abuf_ref, sem_ref, c, slot, t):
    w = min(CW, N - t * CW)
    return pltpu.make_async_copy(
        adj_ref.at[pl.ds(c * CH, CH), pl.ds(t * CW, w)],
        abuf_ref.at[slot, :, pl.ds(t * CW, w)],
        sem_ref.at[slot],
    )


def _gcn_body(
    x_ref, adj_ref, w1_ref, b1_ref, w2_ref, b2_ref, o_ref, s1_ref, s2_ref,
    abuf_ref, sema_ref, semb_ref,
):
    for k in range(K):
        c0 = NCH - 1 - k
        _chunk_copy(adj_ref, abuf_ref, sema_ref, c0, c0 % K).start()

    s1_ref[...] = jnp.dot(x_ref[...], w1_ref[...], preferred_element_type=jnp.float32)
    s2_ref[...] = jnp.zeros((N, H2), jnp.float32)

    def p1_body(it, _):
        c = NCH - 1 - it

        @pl.when(it > 0)
        def _():
            cp = c + 1
            slotp = jax.lax.rem(cp, K)
            o_ref[pl.ds(cp * CH, CH), :] = (
                jnp.dot(
                    abuf_ref[slotp], s2_ref[...], preferred_element_type=jnp.float32
                )
                + b2_ref[...]
            )

            @pl.when(cp >= K)
            def _():
                _chunk_copy(adj_ref, abuf_ref, sema_ref, cp - K, slotp).start()

        slot = jax.lax.rem(c, K)
        _chunk_copy(adj_ref, abuf_ref, sema_ref, c, slot).wait()
        h = jnp.dot(abuf_ref[slot], s1_ref[...], preferred_element_type=jnp.float32)
        h = jnp.maximum(h + b1_ref[...], 0.0)
        s2_ref[pl.ds(c * CH, CH), :] = jnp.dot(
            h, w2_ref[...], preferred_element_type=jnp.float32
        )
        return 0

    jax.lax.fori_loop(0, NCH, p1_body, 0)

    o_ref[pl.ds(0, CH), :] = (
        jnp.dot(abuf_ref[0], s2_ref[...], preferred_element_type=jnp.float32)
        + b2_ref[...]
    )

    row_ids = jax.lax.broadcasted_iota(jnp.int32, (N, H2), 0)

    def fetch_tiles(c, slot):
        ntc = (c * CH + CW - 1) // CW
        for t in range(NT):
            @pl.when(t < ntc)
            def _():
                _tile_copy(adj_ref, abuf_ref, semb_ref, c, slot, t).start()

    def wait_tiles(c, slot):
        ntc = (c * CH + CW - 1) // CW
        for t in range(NT):
            @pl.when(t < ntc)
            def _():
                _tile_copy(adj_ref, abuf_ref, semb_ref, c, slot, t).wait()

    fetch_tiles(K, 0)

    def p2_body(c, _):
        slot = jax.lax.rem(c, K)

        @pl.when(c >= K)
        def _():
            wait_tiles(c, slot)

        s2m = jnp.where(row_ids < c * CH, s2_ref[...], 0.0)
        o_ref[pl.ds(c * CH, CH), :] = o_ref[pl.ds(c * CH, CH), :] + jnp.dot(
            abuf_ref[slot], s2m, preferred_element_type=jnp.float32
        )

        @pl.when(c + K < NCH)
        def _():
            fetch_tiles(c + K, slot)

        return 0

    jax.lax.fori_loop(1, NCH, p2_body, 0)


@jax.jit
def _gcn(x, adj, W1, b1, W2, b2):
    b1r = b1.reshape(1, H1)
    b2r = b2.reshape(1, H2)

    out = pl.pallas_call(
        _gcn_body,
        in_specs=[
            pl.BlockSpec(memory_space=pltpu.MemorySpace.VMEM),
            pl.BlockSpec(memory_space=pl.ANY),
            pl.BlockSpec(memory_space=pltpu.MemorySpace.VMEM),
            pl.BlockSpec(memory_space=pltpu.MemorySpace.VMEM),
            pl.BlockSpec(memory_space=pltpu.MemorySpace.VMEM),
            pl.BlockSpec(memory_space=pltpu.MemorySpace.VMEM),
        ],
        out_specs=pl.BlockSpec(memory_space=pltpu.MemorySpace.VMEM),
        out_shape=jax.ShapeDtypeStruct((N, H2), jnp.float32),
        scratch_shapes=[
            pltpu.VMEM((N, H1), jnp.float32),
            pltpu.VMEM((N, H2), jnp.float32),
            pltpu.VMEM((K, CH, N), jnp.float32),
            pltpu.SemaphoreType.DMA((K,)),
            pltpu.SemaphoreType.DMA((K,)),
        ],
        compiler_params=_PARAMS,
    )(x, adj, W1, b1r, W2, b2r)

    return out


def kernel(x, adj, W1, b1, W2, b2):
    return _gcn(x, adj, W1, b1, W2, b2)
